# baseline (speedup 1.0000x reference)
import functools

import jax
import jax.numpy as jnp
from jax import lax
from jax.experimental import pallas as pl
from jax.experimental.pallas import tpu as pltpu
from jax.experimental.pallas import tpu_sc as plsc

D = 16
B = 16384
V = 1000000
KW = 122880
KMAIN = 8
TAIL = V - KMAIN * KW
KGRID = KMAIN + 1
VOUT = KGRID * KW
NC = 2
NS = 16
NW = NC * NS
BPW = B // NW
CHUNK = 128
NCHUNK = BPW // CHUNK
GROUPS = BPW // 16


def _proj_body(t1_ref, t2_ref, t1tail_ref, t2tail_ref, wt_ref,
               p1_ref, p2_ref):
    i = pl.program_id(0)
    w = wt_ref[...]

    @pl.when(i < KMAIN)
    def _main():
        p1_ref[...] = jnp.sum(t1_ref[...] * w[:, 0:1], axis=0)
        p2_ref[...] = jnp.sum(t2_ref[...] * w[:, 1:2], axis=0)

    @pl.when(i == KMAIN)
    def _tail():
        s = pl.ds(0, TAIL)
        p1_ref[s] = jnp.sum(t1tail_ref[...] * w[:, 0:1], axis=0)
        p2_ref[s] = jnp.sum(t2tail_ref[...] * w[:, 1:2], axis=0)


@jax.jit
def _proj_call(t1t, t2t, t1tail, t2tail, wt):
    return pl.pallas_call(
        _proj_body,
        grid=(KGRID,),
        in_specs=[
            pl.BlockSpec((D, KW), lambda i: (0, jnp.minimum(i, KMAIN - 1))),
            pl.BlockSpec((D, KW), lambda i: (0, jnp.minimum(i, KMAIN - 1))),
            pl.BlockSpec((D, TAIL), lambda i: (0, 0)),
            pl.BlockSpec((D, TAIL), lambda i: (0, 0)),
            pl.BlockSpec((D, 2), lambda i: (0, 0)),
        ],
        out_specs=[
            pl.BlockSpec((KW,), lambda i: (i,)),
            pl.BlockSpec((KW,), lambda i: (i,)),
        ],
        out_shape=[
            jax.ShapeDtypeStruct((VOUT,), jnp.float32),
            jax.ShapeDtypeStruct((VOUT,), jnp.float32),
        ],
        compiler_params=pltpu.CompilerParams(
            dimension_semantics=("parallel",)),
    )(t1t, t2t, t1tail, t2tail, wt)


def _sc_body(rid1_hbm, rid2_hbm, cid1_hbm, cid2_hbm, bb_hbm,
             p1_hbm, p2_hbm, out_hbm,
             rid1_v, rid2_v, cid1_v, cid2_v, rows1_v, rows2_v, bb_v,
             out_v, sem):
    wid = lax.axis_index("s") * NC + lax.axis_index("c")
    row0 = wid * NCHUNK
    base = wid * BPW

    pltpu.sync_copy(rid1_hbm.at[pl.ds(row0, NCHUNK)], rid1_v)
    pltpu.sync_copy(rid2_hbm.at[pl.ds(row0, NCHUNK)], rid2_v)
    pltpu.sync_copy(cid1_hbm.at[pl.ds(base, BPW)], cid1_v)
    pltpu.sync_copy(cid2_hbm.at[pl.ds(base, BPW)], cid2_v)
    pltpu.sync_copy(bb_hbm, bb_v)

    copies = []
    for c in range(NCHUNK):
        copies.append(pltpu.async_copy(
            p1_hbm.at[rid1_v.at[c]], rows1_v.at[pl.ds(c * CHUNK, CHUNK)],
            sem))
        copies.append(pltpu.async_copy(
            p2_hbm.at[rid2_v.at[c]], rows2_v.at[pl.ds(c * CHUNK, CHUNK)],
            sem))
    for cp in copies:
        cp.wait()

    bvec = bb_v[...]

    def group(j, carry):
        s = pl.ds(j * 16, 16)
        rid = j * 16 + jnp.arange(16, dtype=jnp.int32)
        g1 = plsc.load_gather(rows1_v, [rid, cid1_v[s]])
        g2 = plsc.load_gather(rows2_v, [rid, cid2_v[s]])
        out_v[s] = g1 + g2 + bvec
        return carry

    lax.fori_loop(0, GROUPS, group, 0)

    pltpu.sync_copy(out_v, out_hbm.at[pl.ds(base, BPW)])


@jax.jit
def _sc_call(rid1, rid2, cid1, cid2, bb, p1r, p2r):
    mesh = plsc.VectorSubcoreMesh(core_axis_name="c", subcore_axis_name="s")
    k = pl.kernel(
        _sc_body,
        mesh=mesh,
        compiler_params=pltpu.CompilerParams(
            needs_layout_passes=False, use_tc_tiling_on_sc=False),
        out_type=jax.ShapeDtypeStruct((B,), jnp.float32),
        scratch_types=[
            pltpu.VMEM((NCHUNK, CHUNK), jnp.int32),
            pltpu.VMEM((NCHUNK, CHUNK), jnp.int32),
            pltpu.VMEM((BPW,), jnp.int32),
            pltpu.VMEM((BPW,), jnp.int32),
            pltpu.VMEM((BPW, D), jnp.float32),
            pltpu.VMEM((BPW, D), jnp.float32),
            pltpu.VMEM((16,), jnp.float32),
            pltpu.VMEM((BPW,), jnp.float32),
            pltpu.SemaphoreType.DMA,
        ],
    )
    return k(rid1, rid2, cid1, cid2, bb, p1r, p2r)


def kernel(f1, f2, table1, table2, W, b):
    wt = W.reshape(2, D).T
    t1t = table1.T
    t2t = table2.T
    t1tail = lax.slice(t1t, (0, KMAIN * KW), (D, V))
    t2tail = lax.slice(t2t, (0, KMAIN * KW), (D, V))
    p1, p2 = _proj_call(t1t, t2t, t1tail, t2tail, wt)
    rid1 = (f1 >> 4).reshape(128, 128)
    rid2 = (f2 >> 4).reshape(128, 128)
    cid1 = f1 & 15
    cid2 = f2 & 15
    bb = jnp.broadcast_to(b, (16,))
    out = _sc_call(rid1, rid2, cid1, cid2, bb,
                   p1.reshape(VOUT // D, D), p2.reshape(VOUT // D, D))
    return out.reshape(B, 1)


# speedup vs baseline: 1.0379x; 1.0379x over previous
import functools

import jax
import jax.numpy as jnp
from jax import lax
from jax.experimental import pallas as pl
from jax.experimental.pallas import tpu as pltpu
from jax.experimental.pallas import tpu_sc as plsc

D = 16
B = 16384
V = 1000000
KW = 98304
KMAIN = 10
TAIL = V - KMAIN * KW
KGRID = KMAIN + 1
VOUT = KGRID * KW
NC = 2
NS = 16
NW = NC * NS
BPW = B // NW
CHUNK = 128
NCHUNK = BPW // CHUNK
GROUPS = BPW // 16


def _proj_body(t1_ref, t2_ref, t1tail_ref, t2tail_ref, wt_ref,
               p1_ref, p2_ref):
    i = pl.program_id(0)
    w = wt_ref[...]

    @pl.when(i < KMAIN)
    def _main():
        p1_ref[...] = jnp.sum(t1_ref[...] * w[:, 0:1], axis=0)
        p2_ref[...] = jnp.sum(t2_ref[...] * w[:, 1:2], axis=0)

    @pl.when(i == KMAIN)
    def _tail():
        s = pl.ds(0, TAIL)
        p1_ref[s] = jnp.sum(t1tail_ref[...] * w[:, 0:1], axis=0)
        p2_ref[s] = jnp.sum(t2tail_ref[...] * w[:, 1:2], axis=0)


@jax.jit
def _proj_call(t1t, t2t, t1tail, t2tail, wt):
    return pl.pallas_call(
        _proj_body,
        grid=(KGRID,),
        in_specs=[
            pl.BlockSpec((D, KW), lambda i: (0, jnp.minimum(i, KMAIN - 1))),
            pl.BlockSpec((D, KW), lambda i: (0, jnp.minimum(i, KMAIN - 1))),
            pl.BlockSpec((D, TAIL), lambda i: (0, 0)),
            pl.BlockSpec((D, TAIL), lambda i: (0, 0)),
            pl.BlockSpec((D, 2), lambda i: (0, 0)),
        ],
        out_specs=[
            pl.BlockSpec((KW,), lambda i: (i,)),
            pl.BlockSpec((KW,), lambda i: (i,)),
        ],
        out_shape=[
            jax.ShapeDtypeStruct((VOUT,), jnp.float32),
            jax.ShapeDtypeStruct((VOUT,), jnp.float32),
        ],
        compiler_params=pltpu.CompilerParams(
            dimension_semantics=("parallel",)),
    )(t1t, t2t, t1tail, t2tail, wt)


def _sc_body(rid1_hbm, rid2_hbm, cid1_hbm, cid2_hbm, bb_hbm,
             p1_hbm, p2_hbm, out_hbm,
             rid1_v, rid2_v, cid1_v, cid2_v, rows1_v, rows2_v, bb_v,
             out_v, sem):
    wid = lax.axis_index("s") * NC + lax.axis_index("c")
    row0 = wid * NCHUNK
    base = wid * BPW

    stage = [
        pltpu.async_copy(rid1_hbm.at[pl.ds(row0, NCHUNK)], rid1_v, sem),
        pltpu.async_copy(rid2_hbm.at[pl.ds(row0, NCHUNK)], rid2_v, sem),
        pltpu.async_copy(cid1_hbm.at[pl.ds(base, BPW)], cid1_v, sem),
        pltpu.async_copy(cid2_hbm.at[pl.ds(base, BPW)], cid2_v, sem),
        pltpu.async_copy(bb_hbm, bb_v, sem),
    ]
    for cp in stage:
        cp.wait()

    copies = []
    for c in range(NCHUNK):
        copies.append(pltpu.async_copy(
            p1_hbm.at[rid1_v.at[c]], rows1_v.at[pl.ds(c * CHUNK, CHUNK)],
            sem))
        copies.append(pltpu.async_copy(
            p2_hbm.at[rid2_v.at[c]], rows2_v.at[pl.ds(c * CHUNK, CHUNK)],
            sem))
    for cp in copies:
        cp.wait()

    bvec = bb_v[...]

    def group(j, carry):
        s = pl.ds(j * 16, 16)
        rid = j * 16 + jnp.arange(16, dtype=jnp.int32)
        g1 = plsc.load_gather(rows1_v, [rid, cid1_v[s]])
        g2 = plsc.load_gather(rows2_v, [rid, cid2_v[s]])
        out_v[s] = g1 + g2 + bvec
        return carry

    lax.fori_loop(0, GROUPS, group, 0)

    pltpu.sync_copy(out_v, out_hbm.at[pl.ds(base, BPW)])


@jax.jit
def _sc_call(rid1, rid2, cid1, cid2, bb, p1r, p2r):
    mesh = plsc.VectorSubcoreMesh(core_axis_name="c", subcore_axis_name="s")
    k = pl.kernel(
        _sc_body,
        mesh=mesh,
        compiler_params=pltpu.CompilerParams(
            needs_layout_passes=False, use_tc_tiling_on_sc=False),
        out_type=jax.ShapeDtypeStruct((B,), jnp.float32),
        scratch_types=[
            pltpu.VMEM((NCHUNK, CHUNK), jnp.int32),
            pltpu.VMEM((NCHUNK, CHUNK), jnp.int32),
            pltpu.VMEM((BPW,), jnp.int32),
            pltpu.VMEM((BPW,), jnp.int32),
            pltpu.VMEM((BPW, D), jnp.float32),
            pltpu.VMEM((BPW, D), jnp.float32),
            pltpu.VMEM((16,), jnp.float32),
            pltpu.VMEM((BPW,), jnp.float32),
            pltpu.SemaphoreType.DMA,
        ],
    )
    return k(rid1, rid2, cid1, cid2, bb, p1r, p2r)


def kernel(f1, f2, table1, table2, W, b):
    wt = W.reshape(2, D).T
    t1t = table1.T
    t2t = table2.T
    t1tail = lax.slice(t1t, (0, KMAIN * KW), (D, V))
    t2tail = lax.slice(t2t, (0, KMAIN * KW), (D, V))
    p1, p2 = _proj_call(t1t, t2t, t1tail, t2tail, wt)
    rid1 = (f1 >> 4).reshape(128, 128)
    rid2 = (f2 >> 4).reshape(128, 128)
    cid1 = f1 & 15
    cid2 = f2 & 15
    bb = jnp.broadcast_to(b, (16,))
    out = _sc_call(rid1, rid2, cid1, cid2, bb,
                   p1.reshape(VOUT // D, D), p2.reshape(VOUT // D, D))
    return out.reshape(B, 1)
